# baseline (speedup 1.0000x reference)
import functools

import jax
import jax.numpy as jnp
from jax import lax
from jax.experimental import pallas as pl
from jax.experimental.pallas import tpu as pltpu
from jax.experimental.pallas import tpu_sc as plsc

D = 64
NC = 2
NS = 16
NW = NC * NS
CH = 128
NBUF = 4


def _emb_body(idx_hbm, table_hbm, out_hbm, idx_v, rows_v, *gsems):
    nch = idx_hbm.shape[1]
    wid = lax.axis_index("s") * NC + lax.axis_index("c")
    pltpu.sync_copy(idx_hbm.at[wid], idx_v)

    def gather_start(j, b):
        pltpu.make_async_copy(
            table_hbm.at[idx_v.at[j]], rows_v.at[b], gsems[b]).start()

    def gather_wait(j, b):
        pltpu.make_async_copy(
            table_hbm.at[idx_v.at[j]], rows_v.at[b], gsems[b]).wait()

    for b in range(NBUF):
        gather_start(b, b)

    def grp(g, carry):
        for b in range(NBUF):
            j = g * NBUF + b
            gather_wait(j, b)
            pltpu.sync_copy(rows_v.at[b], out_hbm.at[wid, j])

            @pl.when(j + NBUF < nch)
            def _():
                gather_start(j + NBUF, b)
        return carry

    lax.fori_loop(0, nch // NBUF, grp, 0)


@jax.jit
def kernel(token_ids, embeddings):
    bsz, hist = token_ids.shape
    tot = bsz * hist
    nch = tot // (NW * CH)
    idx = token_ids.reshape(NW, nch, CH).astype(jnp.int32)
    run = pl.kernel(
        _emb_body,
        out_type=jax.ShapeDtypeStruct((NW, nch, CH, D), jnp.float32),
        mesh=plsc.VectorSubcoreMesh(
            core_axis_name="c", subcore_axis_name="s",
            num_cores=NC, num_subcores=NS),
        scratch_types=[
            pltpu.VMEM((nch, CH), jnp.int32),
            pltpu.VMEM((NBUF, CH, D), jnp.float32),
        ] + [pltpu.SemaphoreType.DMA] * NBUF,
        compiler_params=pltpu.CompilerParams(use_tc_tiling_on_sc=False),
    )
    out = run(idx, embeddings)
    return out.reshape(bsz, hist, D)
